# baseline (speedup 1.0000x reference)
import math

import jax
import jax.numpy as jnp
from jax import lax
from jax.experimental import pallas as pl
from jax.experimental.pallas import tpu as pltpu
from jax.experimental.pallas import tpu_sc as plsc

D_MODEL = 64
SCALE = math.sqrt(D_MODEL)

NC = 2
NS = 16
NW = NC * NS

BATCH = 1024
SEQ = 200
ROWS_PER_W = BATCH // NW
SPLIT = 128
REM = SEQ - SPLIT
NBUF = 3
NROUND = ROWS_PER_W // NBUF
VIEW_ROWS = 500000


def _emb_kernel(table_hbm, x_hbm, out_hbm, idx_v, idh_v, *scr):
    gbufs = scr[0:NBUF]
    gsems = scr[NBUF:2 * NBUF]
    psems = scr[2 * NBUF:3 * NBUF]

    wid = lax.axis_index("s") * NC + lax.axis_index("c")
    row0 = wid * ROWS_PER_W

    pltpu.sync_copy(x_hbm.at[pl.ds(row0, ROWS_PER_W)], idx_v)

    offs = list(range(0, SEQ - 16 + 1, 16)) + [SEQ - 16]

    def row_shift(r):
        for o in offs:
            sl = pl.ds(o, 16)
            idh_v[r, sl] = lax.shift_right_logical(idx_v[r, sl], 1)

    pl.loop(0, ROWS_PER_W)(row_shift)

    def start_gather(r, b):
        pltpu.async_copy(
            table_hbm.at[idh_v.at[r, pl.ds(0, SPLIT)]],
            gbufs[b].at[pl.ds(0, SPLIT)], gsems[b])
        pltpu.async_copy(
            table_hbm.at[idh_v.at[r, pl.ds(SPLIT, REM)]],
            gbufs[b].at[pl.ds(SPLIT, REM)], gsems[b])

    def wait_gather(b):
        pltpu.make_async_copy(table_hbm.at[idh_v.at[0]], gbufs[b], gsems[b]).wait()

    def start_put(r, b):
        pltpu.async_copy(
            gbufs[b].at[:, pl.ds(0, D_MODEL)],
            out_hbm.at[row0 + r, :, pl.ds(0, D_MODEL)], psems[b])

    def wait_put(b):
        pltpu.make_async_copy(
            gbufs[b].at[:, pl.ds(0, D_MODEL)],
            out_hbm.at[row0, :, pl.ds(0, D_MODEL)], psems[b]).wait()

    def sel_row(r, b):
        gb = gbufs[b]

        def do_lane(pv, t, j):
            off = pv[j]
            for d in range(4):
                src = pl.ds(off + d * 16, 16)
                dst = pl.ds(d * 16, 16)
                gb[t, dst] = gb[t, src] * SCALE

        def window(w):
            t0 = w * 16
            pv = (idx_v[r, pl.ds(t0, 16)] & 1) * D_MODEL
            for j in range(16):
                do_lane(pv, t0 + j, j)

        pl.loop(0, SEQ // 16)(window)

        pvt = (idx_v[r, pl.ds(SEQ - 16, 16)] & 1) * D_MODEL
        for j in range(8):
            do_lane(pvt, SEQ - 8 + j, 8 + j)

    for b in range(NBUF):
        start_gather(b, b)

    for b in range(NBUF):
        wait_gather(b)
        sel_row(b, b)
        start_put(b, b)
        start_gather(b + NBUF, b)

    def round_body(g):
        for b in range(NBUF):
            r = g * NBUF + b
            wait_gather(b)
            wait_put(b)
            sel_row(r, b)
            start_put(r, b)
            start_gather(r + NBUF, b)

    n_full = ROWS_PER_W // NBUF
    rem_rows = ROWS_PER_W - n_full * NBUF
    pl.loop(1, n_full - 1)(round_body)

    for b in range(NBUF):
        r = (n_full - 1) * NBUF + b
        wait_gather(b)
        wait_put(b)
        sel_row(r, b)
        start_put(r, b)
        if r + NBUF < ROWS_PER_W:
            start_gather(r + NBUF, b)

    for k in range(rem_rows):
        r = n_full * NBUF + k
        b = k
        wait_gather(b)
        wait_put(b)
        sel_row(r, b)
        start_put(r, b)

    for b in range(NBUF):
        wait_put(b)


@jax.jit
def kernel(x, table):
    mesh = plsc.VectorSubcoreMesh(core_axis_name="c", subcore_axis_name="s")
    run = pl.kernel(
        _emb_kernel,
        out_type=jax.ShapeDtypeStruct((BATCH, SEQ, 128), jnp.float32),
        mesh=mesh,
        scratch_types=(
            [pltpu.VMEM((ROWS_PER_W, SEQ), jnp.int32),
             pltpu.VMEM((ROWS_PER_W, SEQ), jnp.int32)]
            + [pltpu.VMEM((SEQ, 128), jnp.float32) for _ in range(NBUF)]
            + [pltpu.SemaphoreType.DMA for _ in range(2 * NBUF)]
        ),
        compiler_params=pltpu.CompilerParams(use_tc_tiling_on_sc=False),
    )
    tview = table.reshape(VIEW_ROWS, 128)
    return run(tview, x.astype(jnp.int32))[:, :, :D_MODEL]


# speedup vs baseline: 1.5426x; 1.5426x over previous
import math

import jax
import jax.numpy as jnp
from jax import lax
from jax.experimental import pallas as pl
from jax.experimental.pallas import tpu as pltpu
from jax.experimental.pallas import tpu_sc as plsc

D_MODEL = 64
SCALE = math.sqrt(D_MODEL)

NC = 2
NS = 16
NW = NC * NS

BATCH = 1024
SEQ = 200
ROWS_PER_W = BATCH // NW


def _emb_kernel(table_hbm, x_hbm, out_hbm, idx_v, gbuf, gsem, psem):
    wid = lax.axis_index("s") * NC + lax.axis_index("c")
    row0 = wid * ROWS_PER_W

    pltpu.sync_copy(x_hbm.at[pl.ds(row0, ROWS_PER_W)], idx_v)

    def do_row(r):
        def window(w):
            t0 = w * 16
            v = idx_v[r, pl.ds(t0, 16)]
            for j in range(16):
                pltpu.async_copy(
                    table_hbm.at[pl.ds(v[j], 1)],
                    gbuf.at[pl.ds(t0 + j, 1)], gsem)

        pl.loop(0, SEQ // 16)(window)
        vt = idx_v[r, pl.ds(SEQ - 16, 16)]
        for j in range(8):
            pltpu.async_copy(
                table_hbm.at[pl.ds(vt[8 + j], 1)],
                gbuf.at[pl.ds(SEQ - 8 + j, 1)], gsem)

        pltpu.make_async_copy(table_hbm.at[pl.ds(0, SEQ)], gbuf, gsem).wait()

        def mul_body(t):
            for d in range(4):
                sl = pl.ds(d * 16, 16)
                gbuf[t, sl] = gbuf[t, sl] * SCALE

        pl.loop(0, SEQ, unroll=4)(mul_body)

        pltpu.async_copy(gbuf, out_hbm.at[row0 + r], psem)
        pltpu.make_async_copy(gbuf, out_hbm.at[row0 + r], psem).wait()

    pl.loop(0, ROWS_PER_W)(do_row)


@jax.jit
def kernel(x, table):
    mesh = plsc.VectorSubcoreMesh(core_axis_name="c", subcore_axis_name="s")
    run = pl.kernel(
        _emb_kernel,
        out_type=jax.ShapeDtypeStruct((BATCH, SEQ, D_MODEL), jnp.float32),
        mesh=mesh,
        scratch_types=(
            [pltpu.VMEM((ROWS_PER_W, SEQ), jnp.int32),
             pltpu.VMEM((SEQ, D_MODEL), jnp.float32),
             pltpu.SemaphoreType.DMA,
             pltpu.SemaphoreType.DMA]
        ),
        compiler_params=pltpu.CompilerParams(use_tc_tiling_on_sc=True),
    )
    return run(table, x.astype(jnp.int32))
